# baseline (speedup 1.0000x reference)
import jax
import jax.numpy as jnp
from jax.experimental import pallas as pl

IMG = 224; PATCH = 16; C_IN = 3; EMBED = 512; NEXP = 10; TOPK = 2; NHEAD = 8; BATCH = 8
NTOK = (IMG // PATCH) ** 2
PDIM = PATCH * PATCH * C_IN
HDIM = 4 * EMBED
HD = EMBED // NHEAD
R = BATCH * NTOK
TTILE = 224
NTILE = R // TTILE

_F32 = jnp.float32


def _dot(a, b, dims):
    return jax.lax.dot_general(a, b, (dims, ((), ())),
                               preferred_element_type=_F32)


def _ln_rows(x, g, b, eps=1e-5):
    m = jnp.mean(x, axis=-1, keepdims=True)
    v = jnp.mean((x - m) ** 2, axis=-1, keepdims=True)
    return (x - m) / jnp.sqrt(v + eps) * g + b


def _embed_attn_body(xp_ref, wp_ref, bp_ref, g1_ref, b1_ref,
                     wq_ref, wk_ref, wv_ref, wo_ref, bo_ref, pos_ref, t_ref):
    x = xp_ref[0]
    t0 = _dot(x, wp_ref[...], ((1,), (1,))) + bp_ref[...]
    ln = _ln_rows(t0, g1_ref[...], b1_ref[...])
    q = _dot(ln, wq_ref[...], ((1,), (1,)))
    k = _dot(ln, wk_ref[...], ((1,), (1,)))
    v = _dot(ln, wv_ref[...], ((1,), (1,)))
    heads = []
    for h in range(NHEAD):
        sl = slice(h * HD, (h + 1) * HD)
        att = _dot(q[:, sl], k[:, sl], ((1,), (1,))) * (HD ** -0.5)
        att = jax.nn.softmax(att, axis=-1)
        heads.append(_dot(att, v[:, sl], ((1,), (0,))))
    o = jnp.concatenate(heads, axis=-1)
    o = _dot(o, wo_ref[...], ((1,), (1,))) + bo_ref[...]
    t_ref[0] = t0 + o + pos_ref[0]


def _embed_attn(xp, p):
    return pl.pallas_call(
        _embed_attn_body,
        grid=(BATCH,),
        in_specs=[
            pl.BlockSpec((1, NTOK, PDIM), lambda b: (b, 0, 0)),
            pl.BlockSpec((EMBED, PDIM), lambda b: (0, 0)),
            pl.BlockSpec((1, EMBED), lambda b: (0, 0)),
            pl.BlockSpec((1, EMBED), lambda b: (0, 0)),
            pl.BlockSpec((1, EMBED), lambda b: (0, 0)),
            pl.BlockSpec((EMBED, EMBED), lambda b: (0, 0)),
            pl.BlockSpec((EMBED, EMBED), lambda b: (0, 0)),
            pl.BlockSpec((EMBED, EMBED), lambda b: (0, 0)),
            pl.BlockSpec((EMBED, EMBED), lambda b: (0, 0)),
            pl.BlockSpec((1, EMBED), lambda b: (0, 0)),
            pl.BlockSpec((1, NTOK, EMBED), lambda b: (0, 0, 0)),
        ],
        out_specs=pl.BlockSpec((1, NTOK, EMBED), lambda b: (b, 0, 0)),
        out_shape=jax.ShapeDtypeStruct((BATCH, NTOK, EMBED), _F32),
    )(xp, p['Wp'], p['bp'].reshape(1, -1), p['g1'].reshape(1, -1),
      p['bln1'].reshape(1, -1), p['Wq'], p['Wk'], p['Wv'], p['Wo'],
      p['bo'].reshape(1, -1), p['pos'])


def _router_body(t_ref, g_ref, b_ref, wt_ref, bt_ref, wn_ref, bn_ref,
                 noise_ref, xln_ref, gate_ref):
    x = _ln_rows(t_ref[...], g_ref[...], b_ref[...])
    logits = _dot(x, wt_ref[...], ((1,), (1,))) + bt_ref[...]
    nl = _dot(x, wn_ref[...], ((1,), (1,))) + bn_ref[...]
    noisy = logits + noise_ref[...] * jax.nn.softplus(nl)
    m1 = jnp.max(noisy, axis=-1, keepdims=True)
    ninf = jnp.float32(-jnp.inf)
    m2 = jnp.max(jnp.where(noisy == m1, ninf, noisy), axis=-1, keepdims=True)
    sel = noisy >= m2
    e = jnp.where(sel, jnp.exp(noisy - m1), 0.0)
    gate_ref[...] = e / jnp.sum(e, axis=-1, keepdims=True)
    xln_ref[...] = x


def _router(t_flat, mp, g, b, noise):
    full = lambda *s: pl.BlockSpec(s, lambda: tuple(0 for _ in s))
    return pl.pallas_call(
        _router_body,
        in_specs=[
            full(R, EMBED), full(1, EMBED), full(1, EMBED),
            full(NEXP, EMBED), full(1, NEXP),
            full(NEXP, EMBED), full(1, NEXP), full(R, NEXP),
        ],
        out_specs=[full(R, EMBED), full(R, NEXP)],
        out_shape=[jax.ShapeDtypeStruct((R, EMBED), _F32),
                   jax.ShapeDtypeStruct((R, NEXP), _F32)],
    )(t_flat, g.reshape(1, -1), b.reshape(1, -1),
      mp['Wt'], mp['bt'].reshape(1, -1), mp['Wn'], mp['bn'].reshape(1, -1),
      noise)


def _moe_body(x_ref, w1_ref, b1_ref, w2_ref, b2_ref, gt_ref, eo_ref, f_ref):
    e = pl.program_id(0)
    ti = pl.program_id(1)
    x = x_ref[...]
    h = jnp.maximum(_dot(x, w1_ref[0], ((1,), (1,))) + b1_ref[0], 0.0)
    o = _dot(h, w2_ref[0], ((1,), (1,))) + b2_ref[0]
    gate = gt_ref[0, ti][:, None]
    w = o * gate
    eo_ref[0] = w
    rows = pl.ds(ti * TTILE, TTILE)

    @pl.when(e == 0)
    def _():
        f_ref[rows, :] = w

    @pl.when(e > 0)
    def _():
        f_ref[rows, :] = f_ref[rows, :] + w


def _moe_dense(xln, gating, mp):
    gate_t = gating.T.reshape(NEXP, NTILE, TTILE)
    eo, f = pl.pallas_call(
        _moe_body,
        grid=(NEXP, NTILE),
        in_specs=[
            pl.BlockSpec((TTILE, EMBED), lambda e, ti: (ti, 0)),
            pl.BlockSpec((1, HDIM, EMBED), lambda e, ti: (e, 0, 0)),
            pl.BlockSpec((1, 1, HDIM), lambda e, ti: (e, 0, 0)),
            pl.BlockSpec((1, EMBED, HDIM), lambda e, ti: (e, 0, 0)),
            pl.BlockSpec((1, 1, EMBED), lambda e, ti: (e, 0, 0)),
            pl.BlockSpec((1, NTILE, TTILE), lambda e, ti: (e, 0, 0)),
        ],
        out_specs=[
            pl.BlockSpec((1, TTILE, EMBED), lambda e, ti: (e, ti, 0)),
            pl.BlockSpec((R, EMBED), lambda e, ti: (0, 0)),
        ],
        out_shape=[jax.ShapeDtypeStruct((NEXP, R, EMBED), _F32),
                   jax.ShapeDtypeStruct((R, EMBED), _F32)],
    )(xln, mp['W1'], mp['b1'].reshape(NEXP, 1, HDIM),
      mp['W2'], mp['b2'].reshape(NEXP, 1, EMBED), gate_t)
    return f, eo


def _head_body(f2_ref, wc_ref, bc_ref, feat_ref, cls_ref):
    feat = jnp.mean(f2_ref[...], axis=1)
    feat_ref[...] = feat
    cls_ref[...] = _dot(feat, wc_ref[...], ((1,), (1,))) + bc_ref[...]


def _head(f2, wc, bc):
    full = lambda *s: pl.BlockSpec(s, lambda: tuple(0 for _ in s))
    return pl.pallas_call(
        _head_body,
        in_specs=[full(BATCH, NTOK, EMBED), full(NEXP, EMBED), full(1, NEXP)],
        out_specs=[full(BATCH, EMBED), full(BATCH, NEXP)],
        out_shape=[jax.ShapeDtypeStruct((BATCH, EMBED), _F32),
                   jax.ShapeDtypeStruct((BATCH, NEXP), _F32)],
    )(f2, wc, bc.reshape(1, -1))


def kernel(x, params):
    b, c, h, w = x.shape
    xp = x.reshape(b, c, h // PATCH, PATCH, w // PATCH, PATCH)
    xp = xp.transpose(0, 1, 2, 4, 3, 5).reshape(b, c, -1, PATCH * PATCH)
    xp = xp.transpose(0, 2, 1, 3).reshape(b, -1, PDIM)

    t = _embed_attn(xp, params)
    t_flat = t.reshape(R, EMBED)

    noise1 = jax.random.normal(jax.random.key(1), (BATCH, NTOK, NEXP),
                               dtype=_F32).reshape(R, NEXP)
    noise2 = jax.random.normal(jax.random.key(2), (BATCH, NTOK, NEXP),
                               dtype=_F32).reshape(R, NEXP)

    xln1, gate1 = _router(t_flat, params['moe1'], params['g2'],
                          params['bln2'], noise1)
    xln2, gate2 = _router(t_flat, params['moe2'], params['g3'],
                          params['bln3'], noise2)

    f1_flat, eo1 = _moe_dense(xln1, gate1, params['moe1'])
    f2_flat, eo2 = _moe_dense(xln2, gate2, params['moe2'])

    f1 = f1_flat.reshape(BATCH, NTOK, EMBED)
    f2 = f2_flat.reshape(BATCH, NTOK, EMBED)
    e1 = eo1.reshape(NEXP, BATCH, NTOK, EMBED)
    e2 = eo2.reshape(NEXP, BATCH, NTOK, EMBED)
    gt1 = gate1.reshape(BATCH, NTOK, NEXP)
    gt2 = gate2.reshape(BATCH, NTOK, NEXP)

    feat, cls = _head(f2, params['Wc'], params['bc'])
    return (f1, f2, feat, cls, e1, e2, gt1, gt2)
